# baseline (speedup 1.0000x reference)
import functools

import jax
import jax.numpy as jnp
from jax import lax
from jax.experimental import pallas as pl
from jax.experimental.pallas import tpu as pltpu
from jax.experimental.pallas import tpu_sc as plsc

N_NODES = 10000
D = 128
N_EDGES = 320000

NC = 2
NS = 16
NW = NC * NS

RANGE = 312
TAIL = N_NODES - NW * RANGE
STRIPE = N_EDGES // NS
VREGS = STRIPE // 16
CHUNK = 80
STAGE_CAP = STRIPE + 32 * (CHUNK - 1) // 8 * 8 + 32 * 8
STAGE_CAP = 22560
PAIRS = NS * STAGE_CAP
M312 = 13444
M80 = 52429
ZROW = N_NODES
IOTA = None


def _lane_bcast(v, l):
  return _vgather(v, jnp.full((16,), l, jnp.int32))


def _rank_hist(b_v, iota):
  rank = jnp.zeros((16,), jnp.int32)
  hlo = jnp.zeros((16,), jnp.int32)
  hhi = jnp.zeros((16,), jnp.int32)
  one = jnp.ones((16,), jnp.int32)
  zero = jnp.zeros((16,), jnp.int32)
  for m in range(16):
    bm = _lane_bcast(b_v, m)
    eq_i = jnp.where(b_v == bm, one, zero)
    rank = rank + jnp.where(iota > m, eq_i, zero)
    hlo = hlo + jnp.where(iota == bm, one, zero)
    hhi = hhi + jnp.where(iota == bm - 16, one, zero)
  return rank, hlo, hhi


def _vgather(v, idx):
  dnums = lax.GatherDimensionNumbers(
      offset_dims=(), collapsed_slice_dims=(0,), start_index_map=(0,))
  return lax.gather(v, idx.reshape(16, 1), dnums, (1,),
                    mode=lax.GatherScatterMode.PROMISE_IN_BOUNDS)


def _cumsum16(v, iota):
  zero = jnp.zeros((16,), jnp.int32)
  for sh in (1, 2, 4, 8):
    shifted = _vgather(v, jnp.maximum(iota - sh, 0))
    v = v + jnp.where(iota >= sh, shifted, zero)
  return v


def _wp_lookup(wp0, wp1, b_v):
  glo = _vgather(wp0, jnp.bitwise_and(b_v, 15))
  ghi = _vgather(wp1, jnp.bitwise_and(b_v, 15))
  return jnp.where(b_v < 16, glo, ghi)


def _bin_edges(src, dst):
  mesh = plsc.VectorSubcoreMesh(core_axis_name="c", subcore_axis_name="s")

  @functools.partial(
      pl.kernel,
      out_type=(
          jax.ShapeDtypeStruct((PAIRS,), jnp.int32),
          jax.ShapeDtypeStruct((PAIRS,), jnp.int32),
          jax.ShapeDtypeStruct((NS * 32,), jnp.int32),
          jax.ShapeDtypeStruct((NS * 32,), jnp.int32),
      ),
      mesh=mesh,
      scratch_types=[
          pltpu.VMEM((STRIPE,), jnp.int32),
          pltpu.VMEM((STRIPE,), jnp.int32),
          pltpu.VMEM((CHUNK,), jnp.int32),
          pltpu.VMEM((CHUNK,), jnp.int32),
          pltpu.VMEM((CHUNK,), jnp.int32),
          pltpu.VMEM((32,), jnp.int32),
          pltpu.VMEM((32,), jnp.int32),
          pltpu.VMEM((STAGE_CAP,), jnp.int32),
          pltpu.VMEM_SHARED((PAIRS + CHUNK,), jnp.int32),
          pltpu.VMEM_SHARED((PAIRS + CHUNK,), jnp.int32),
      ],
  )
  def k(src_hbm, dst_hbm, srcb_hbm, dstb_hbm, bases_hbm, trips_hbm,
        sv_m, dv_m, pos_m, zs_m, zd_m, bb_m, tb_m, bnc_m, ssb_sp, dsb_sp):
    c = lax.axis_index("c")
    t = lax.axis_index("s")

    @pl.when(c == 0)
    def _():
      iota = lax.iota(jnp.int32, 16)
      pltpu.sync_copy(src_hbm.at[pl.ds(t * STRIPE, STRIPE)], sv_m)
      pltpu.sync_copy(dst_hbm.at[pl.ds(t * STRIPE, STRIPE)], dv_m)

      def pass_a(v, carry):
        h0, h1 = carry
        d_v = dv_m[pl.ds(v * 16, 16)]
        b_v = jnp.minimum(lax.shift_right_logical(d_v * M312, 22), 31)
        _, hlo, hhi = _rank_hist(b_v, iota)
        return (h0 + hlo, h1 + hhi)

      z16 = jnp.zeros((16,), jnp.int32)
      c0, c1 = lax.fori_loop(0, VREGS, pass_a, (z16, z16), unroll=False)

      tr0 = lax.shift_right_logical((c0 + 79) * M80, 22)
      tr1 = lax.shift_right_logical((c1 + 79) * M80, 22)
      p0 = tr0 * CHUNK
      p1 = tr1 * CHUNK
      cs0 = _cumsum16(p0, iota)
      ex0 = cs0 - p0
      ex1 = _cumsum16(p1, iota) - p1 + _lane_bcast(cs0, 15)
      gbase = t * STAGE_CAP
      bb_m[pl.ds(0, 16)] = ex0 + gbase
      bb_m[pl.ds(16, 16)] = ex1 + gbase
      tb_m[pl.ds(0, 16)] = tr0
      tb_m[pl.ds(16, 16)] = tr1
      pltpu.sync_copy(bb_m, bases_hbm.at[pl.ds(t * 32, 32)])
      pltpu.sync_copy(tb_m, trips_hbm.at[pl.ds(t * 32, 32)])

      zs = jnp.full((16,), ZROW, jnp.int32)
      zd = jnp.zeros((16,), jnp.int32)
      for kk in range(CHUNK // 16):
        zs_m[pl.ds(kk * 16, 16)] = zs
        zd_m[pl.ds(kk * 16, 16)] = zd

      gb0 = ex0 + gbase
      gb1 = ex1 + gbase

      def pass_b(g, carry):
        wp0, wp1 = carry
        for kk in range(CHUNK // 16):
          d_v = dv_m[pl.ds(g * CHUNK + kk * 16, 16)]
          b_v = jnp.minimum(lax.shift_right_logical(d_v * M312, 22), 31)
          rank, hlo, hhi = _rank_hist(b_v, iota)
          pos_m[pl.ds(kk * 16, 16)] = _wp_lookup(wp0, wp1, b_v) + rank
          wp0 = wp0 + hlo
          wp1 = wp1 + hhi
        pltpu.sync_copy(sv_m.at[pl.ds(g * CHUNK, CHUNK)], ssb_sp.at[pos_m])
        pltpu.sync_copy(dv_m.at[pl.ds(g * CHUNK, CHUNK)], dsb_sp.at[pos_m])
        return (wp0, wp1)

      wpf0, wpf1 = lax.fori_loop(0, STRIPE // CHUNK, pass_b, (gb0, gb1),
                                 unroll=False)

      end0 = gb0 + p0
      end1 = gb1 + p1
      for b in range(32):
        wpb = _lane_bcast(wpf0 if b < 16 else wpf1, b & 15)
        endb = _lane_bcast(end0 if b < 16 else end1, b & 15)
        for kk in range(CHUNK // 16):
          p = wpb + iota + kk * 16
          pos_m[pl.ds(kk * 16, 16)] = jnp.where(
              p < endb, p, PAIRS + iota + kk * 16)
        pltpu.sync_copy(zs_m, ssb_sp.at[pos_m])
        pltpu.sync_copy(zd_m, dsb_sp.at[pos_m])

      pltpu.sync_copy(ssb_sp.at[pl.ds(t * STAGE_CAP, STAGE_CAP)], bnc_m)
      pltpu.sync_copy(bnc_m, srcb_hbm.at[pl.ds(t * STAGE_CAP, STAGE_CAP)])
      pltpu.sync_copy(dsb_sp.at[pl.ds(t * STAGE_CAP, STAGE_CAP)], bnc_m)
      pltpu.sync_copy(bnc_m, dstb_hbm.at[pl.ds(t * STAGE_CAP, STAGE_CAP)])

  return k(src, dst)


def _sc_scatter(h_pad, srcb, dstb, bases, trips, zeros):
  mesh = plsc.VectorSubcoreMesh(core_axis_name="c", subcore_axis_name="s")

  @functools.partial(
      pl.kernel,
      out_type=jax.ShapeDtypeStruct((N_NODES, D), jnp.float32),
      mesh=mesh,
      scratch_types=[
          pltpu.VMEM((NS * 32,), jnp.int32),
          pltpu.VMEM((NS * 32,), jnp.int32),
          pltpu.VMEM((CHUNK,), jnp.int32),
          pltpu.VMEM((CHUNK,), jnp.int32),
          pltpu.VMEM((CHUNK, D), jnp.float32),
          pltpu.VMEM_SHARED((N_NODES, D), jnp.float32),
          pltpu.SemaphoreType.DMA,
      ],
  )
  def k(h_hbm, srcb_hbm, dstb_hbm, bases_hbm, trips_hbm, zero_hbm, out_hbm,
        bt_m, tt_m, si_m, di_m, rows_m, agg_sh, sem):
    c = lax.axis_index("c")
    s = lax.axis_index("s")
    wid = c * NS + s
    iota = lax.iota(jnp.int32, 16)

    pltpu.sync_copy(bases_hbm, bt_m)
    pltpu.sync_copy(trips_hbm, tt_m)

    r0 = wid * RANGE
    pltpu.sync_copy(zero_hbm.at[pl.ds(r0, RANGE)],
                    agg_sh.at[pl.ds(r0, RANGE)])

    @pl.when(wid == NW - 1)
    def _():
      pltpu.sync_copy(zero_hbm.at[pl.ds(NW * RANGE, TAIL)],
                      agg_sh.at[pl.ds(NW * RANGE, TAIL)])

    plsc.subcore_barrier()

    toff = pl.multiple_of(wid * NS, 8)
    bvec = bt_m[pl.ds(toff, 16)]
    tvec = tt_m[pl.ds(toff, 16)]
    for t in range(NS):
      base_t = bvec[t]
      trips_t = tvec[t]

      def run(g, _):
        cb = pl.multiple_of(base_t + g * CHUNK, 8)
        pltpu.sync_copy(srcb_hbm.at[pl.ds(cb, CHUNK)], si_m)
        pltpu.sync_copy(dstb_hbm.at[pl.ds(cb, CHUNK)], di_m)
        pltpu.async_copy(h_hbm.at[si_m], rows_m, sem).wait()
        pltpu.sync_copy(rows_m, agg_sh.at[di_m], add=True)
        return ()

      lax.fori_loop(0, trips_t, run, (), unroll=False)

    plsc.subcore_barrier()
    pltpu.sync_copy(agg_sh.at[pl.ds(r0, RANGE)],
                    out_hbm.at[pl.ds(r0, RANGE)])

    @pl.when(wid == NW - 1)
    def _():
      pltpu.sync_copy(agg_sh.at[pl.ds(NW * RANGE, TAIL)],
                      out_hbm.at[pl.ds(NW * RANGE, TAIL)])

  return k(h_pad, srcb, dstb, bases, trips, zeros)


_BLK = 2000
_PREC = lax.Precision.DEFAULT


def _dot(a, b):
  return lax.dot_general(a, b, (((1,), (0,)), ((), ())), precision=_PREC,
                         preferred_element_type=jnp.float32)


def _in_proj(x, w, bias):
  def body(x_ref, w_ref, b_ref, o_ref):
    o_ref[...] = _dot(x_ref[...], w_ref[...]) + b_ref[...]

  return pl.pallas_call(
      body,
      grid=(N_NODES // _BLK,),
      in_specs=[
          pl.BlockSpec((_BLK, D), lambda i: (i, 0)),
          pl.BlockSpec((D, D), lambda i: (0, 0)),
          pl.BlockSpec((1, D), lambda i: (0, 0)),
      ],
      out_specs=pl.BlockSpec((_BLK, D), lambda i: (i, 0)),
      out_shape=jax.ShapeDtypeStruct((N_NODES, D), jnp.float32),
  )(x, w, bias)


def _gin_update(h, agg, w, bias):
  def body(h_ref, a_ref, w_ref, b_ref, o_ref):
    o_ref[...] = _dot(h_ref[...] + a_ref[...], w_ref[...]) + b_ref[...]

  return pl.pallas_call(
      body,
      grid=(N_NODES // _BLK,),
      in_specs=[
          pl.BlockSpec((_BLK, D), lambda i: (i, 0)),
          pl.BlockSpec((_BLK, D), lambda i: (i, 0)),
          pl.BlockSpec((D, D), lambda i: (0, 0)),
          pl.BlockSpec((1, D), lambda i: (0, 0)),
      ],
      out_specs=pl.BlockSpec((_BLK, D), lambda i: (i, 0)),
      out_shape=jax.ShapeDtypeStruct((N_NODES, D), jnp.float32),
  )(h, agg, w, bias)


def _out_proj(h0, h1, h2, h3, w, bias):
  def body(h0_ref, h1_ref, h2_ref, h3_ref, w_ref, b_ref, o_ref):
    cat = jnp.concatenate(
        [h0_ref[...], h1_ref[...], h2_ref[...], h3_ref[...]], axis=1)
    logits = _dot(cat, w_ref[...]) + b_ref[...]
    m = jnp.max(logits, axis=-1, keepdims=True)
    e = jnp.exp(logits - m)
    o_ref[...] = e / jnp.sum(e, axis=-1, keepdims=True)

  hspec = pl.BlockSpec((_BLK, D), lambda i: (i, 0))
  return pl.pallas_call(
      body,
      grid=(N_NODES // _BLK,),
      in_specs=[
          hspec, hspec, hspec, hspec,
          pl.BlockSpec((4 * D, D), lambda i: (0, 0)),
          pl.BlockSpec((1, D), lambda i: (0, 0)),
      ],
      out_specs=hspec,
      out_shape=jax.ShapeDtypeStruct((N_NODES, D), jnp.float32),
  )(h0, h1, h2, h3, w, bias)


def kernel(x, edge_index, W_in, b_in, W1, b1, W2, b2, W3, b3, W_out, b_out):
  src = edge_index[0].astype(jnp.int32)
  dst = edge_index[1].astype(jnp.int32)
  zeros = jnp.zeros((N_NODES, D), jnp.float32)
  zrows = jnp.zeros((8, D), jnp.float32)

  srcb, dstb, bases, trips = _bin_edges(src, dst)
  bases = bases.reshape(NS, 32).T.reshape(-1)
  trips = trips.reshape(NS, 32).T.reshape(-1)

  h = _in_proj(x, W_in, b_in.reshape(1, D))
  hs = [h]
  for w, bias in ((W1, b1), (W2, b2), (W3, b3)):
    h_pad = jnp.concatenate([h, zrows], axis=0)
    agg = _sc_scatter(h_pad, srcb, dstb, bases, trips, zeros)
    h = _gin_update(h, agg, w, bias.reshape(1, D))
    hs.append(h)

  return _out_proj(hs[0], hs[1], hs[2], hs[3], W_out, b_out.reshape(1, D))


# speedup vs baseline: 1.0119x; 1.0119x over previous
import functools

import jax
import jax.numpy as jnp
from jax import lax
from jax.experimental import pallas as pl
from jax.experimental.pallas import tpu as pltpu
from jax.experimental.pallas import tpu_sc as plsc

N_NODES = 10000
D = 128
N_EDGES = 320000

NC = 2
NS = 16
NW = NC * NS

RANGE = 312
TAIL = N_NODES - NW * RANGE
STRIPE = N_EDGES // NS
VREGS = STRIPE // 16
CHUNK = 80
STAGE_CAP = STRIPE + 32 * (CHUNK - 1) // 8 * 8 + 32 * 8
STAGE_CAP = 22560
PAIRS = NS * STAGE_CAP
M312 = 13444
M80 = 52429
ZROW = N_NODES
IOTA = None


def _lane_bcast(v, l):
  return _vgather(v, jnp.full((16,), l, jnp.int32))


def _rank_hist(b_v, iota):
  rank = jnp.zeros((16,), jnp.int32)
  hlo = jnp.zeros((16,), jnp.int32)
  hhi = jnp.zeros((16,), jnp.int32)
  one = jnp.ones((16,), jnp.int32)
  zero = jnp.zeros((16,), jnp.int32)
  for m in range(16):
    bm = _lane_bcast(b_v, m)
    eq_i = jnp.where(b_v == bm, one, zero)
    rank = rank + jnp.where(iota > m, eq_i, zero)
    hlo = hlo + jnp.where(iota == bm, one, zero)
    hhi = hhi + jnp.where(iota == bm - 16, one, zero)
  return rank, hlo, hhi


def _vgather(v, idx):
  dnums = lax.GatherDimensionNumbers(
      offset_dims=(), collapsed_slice_dims=(0,), start_index_map=(0,))
  return lax.gather(v, idx.reshape(16, 1), dnums, (1,),
                    mode=lax.GatherScatterMode.PROMISE_IN_BOUNDS)


def _cumsum16(v, iota):
  zero = jnp.zeros((16,), jnp.int32)
  for sh in (1, 2, 4, 8):
    shifted = _vgather(v, jnp.maximum(iota - sh, 0))
    v = v + jnp.where(iota >= sh, shifted, zero)
  return v


def _wp_lookup(wp0, wp1, b_v):
  glo = _vgather(wp0, jnp.bitwise_and(b_v, 15))
  ghi = _vgather(wp1, jnp.bitwise_and(b_v, 15))
  return jnp.where(b_v < 16, glo, ghi)


def _bin_edges(src, dst):
  mesh = plsc.VectorSubcoreMesh(core_axis_name="c", subcore_axis_name="s")

  @functools.partial(
      pl.kernel,
      out_type=(
          jax.ShapeDtypeStruct((PAIRS,), jnp.int32),
          jax.ShapeDtypeStruct((PAIRS,), jnp.int32),
          jax.ShapeDtypeStruct((NS * 32,), jnp.int32),
          jax.ShapeDtypeStruct((NS * 32,), jnp.int32),
      ),
      mesh=mesh,
      scratch_types=[
          pltpu.VMEM((STRIPE,), jnp.int32),
          pltpu.VMEM((STRIPE,), jnp.int32),
          pltpu.VMEM((CHUNK,), jnp.int32),
          pltpu.VMEM((CHUNK,), jnp.int32),
          pltpu.VMEM((CHUNK,), jnp.int32),
          pltpu.VMEM((CHUNK,), jnp.int32),
          pltpu.SemaphoreType.DMA,
          pltpu.SemaphoreType.DMA,
          pltpu.VMEM((32,), jnp.int32),
          pltpu.VMEM((32,), jnp.int32),
          pltpu.VMEM((STAGE_CAP,), jnp.int32),
          pltpu.VMEM_SHARED((PAIRS + CHUNK,), jnp.int32),
          pltpu.VMEM_SHARED((PAIRS + CHUNK,), jnp.int32),
      ],
  )
  def k(src_hbm, dst_hbm, srcb_hbm, dstb_hbm, bases_hbm, trips_hbm,
        sv_m, dv_m, pos_a, pos_b, zs_m, zd_m, sem_a, sem_b, bb_m, tb_m,
        bnc_m, ssb_sp, dsb_sp):
    c = lax.axis_index("c")
    t = lax.axis_index("s")

    @pl.when(c == 0)
    def _():
      iota = lax.iota(jnp.int32, 16)
      pltpu.sync_copy(src_hbm.at[pl.ds(t * STRIPE, STRIPE)], sv_m)
      pltpu.sync_copy(dst_hbm.at[pl.ds(t * STRIPE, STRIPE)], dv_m)

      def pass_a(v, carry):
        h0, h1 = carry
        d_v = dv_m[pl.ds(v * 16, 16)]
        b_v = jnp.minimum(lax.shift_right_logical(d_v * M312, 22), 31)
        _, hlo, hhi = _rank_hist(b_v, iota)
        return (h0 + hlo, h1 + hhi)

      z16 = jnp.zeros((16,), jnp.int32)
      c0, c1 = lax.fori_loop(0, VREGS, pass_a, (z16, z16), unroll=False)

      tr0 = lax.shift_right_logical((c0 + 79) * M80, 22)
      tr1 = lax.shift_right_logical((c1 + 79) * M80, 22)
      p0 = tr0 * CHUNK
      p1 = tr1 * CHUNK
      cs0 = _cumsum16(p0, iota)
      ex0 = cs0 - p0
      ex1 = _cumsum16(p1, iota) - p1 + _lane_bcast(cs0, 15)
      gbase = t * STAGE_CAP
      bb_m[pl.ds(0, 16)] = ex0 + gbase
      bb_m[pl.ds(16, 16)] = ex1 + gbase
      tb_m[pl.ds(0, 16)] = tr0
      tb_m[pl.ds(16, 16)] = tr1
      pltpu.sync_copy(bb_m, bases_hbm.at[pl.ds(t * 32, 32)])
      pltpu.sync_copy(tb_m, trips_hbm.at[pl.ds(t * 32, 32)])

      zs = jnp.full((16,), ZROW, jnp.int32)
      zd = jnp.zeros((16,), jnp.int32)
      for kk in range(CHUNK // 16):
        zs_m[pl.ds(kk * 16, 16)] = zs
        zd_m[pl.ds(kk * 16, 16)] = zd

      gb0 = ex0 + gbase
      gb1 = ex1 + gbase

      def pass_b(gi, carry):
        wp0, wp1 = carry
        waits = []
        for hh, (pbuf, sem) in enumerate(((pos_a, sem_a), (pos_b, sem_b))):
          g = gi * 2 + hh
          for kk in range(CHUNK // 16):
            d_v = dv_m[pl.ds(g * CHUNK + kk * 16, 16)]
            b_v = jnp.minimum(lax.shift_right_logical(d_v * M312, 22), 31)
            rank, hlo, hhi = _rank_hist(b_v, iota)
            pbuf[pl.ds(kk * 16, 16)] = _wp_lookup(wp0, wp1, b_v) + rank
            wp0 = wp0 + hlo
            wp1 = wp1 + hhi
          waits.append(pltpu.async_copy(
              sv_m.at[pl.ds(g * CHUNK, CHUNK)], ssb_sp.at[pbuf], sem))
          waits.append(pltpu.async_copy(
              dv_m.at[pl.ds(g * CHUNK, CHUNK)], dsb_sp.at[pbuf], sem))
        for w in waits:
          w.wait()
        return (wp0, wp1)

      wpf0, wpf1 = lax.fori_loop(0, STRIPE // CHUNK // 2, pass_b,
                                 (gb0, gb1), unroll=False)

      end0 = gb0 + p0
      end1 = gb1 + p1
      pend = []
      for b in range(32):
        if len(pend) == 4:
          for w in pend[:2]:
            w.wait()
          pend = pend[2:]
        pbuf, sem = ((pos_a, sem_a) if b % 2 == 0 else (pos_b, sem_b))
        wpb = _lane_bcast(wpf0 if b < 16 else wpf1, b & 15)
        endb = _lane_bcast(end0 if b < 16 else end1, b & 15)
        for kk in range(CHUNK // 16):
          p = wpb + iota + kk * 16
          pbuf[pl.ds(kk * 16, 16)] = jnp.where(
              p < endb, p, PAIRS + iota + kk * 16)
        pend.append(pltpu.async_copy(zs_m, ssb_sp.at[pbuf], sem))
        pend.append(pltpu.async_copy(zd_m, dsb_sp.at[pbuf], sem))
      for w in pend:
        w.wait()

      pltpu.sync_copy(ssb_sp.at[pl.ds(t * STAGE_CAP, STAGE_CAP)], bnc_m)
      pltpu.sync_copy(bnc_m, srcb_hbm.at[pl.ds(t * STAGE_CAP, STAGE_CAP)])
      pltpu.sync_copy(dsb_sp.at[pl.ds(t * STAGE_CAP, STAGE_CAP)], bnc_m)
      pltpu.sync_copy(bnc_m, dstb_hbm.at[pl.ds(t * STAGE_CAP, STAGE_CAP)])

  return k(src, dst)


def _sc_scatter(h_pad, srcb, dstb, bases, trips, zeros):
  mesh = plsc.VectorSubcoreMesh(core_axis_name="c", subcore_axis_name="s")

  @functools.partial(
      pl.kernel,
      out_type=jax.ShapeDtypeStruct((N_NODES, D), jnp.float32),
      mesh=mesh,
      scratch_types=[
          pltpu.VMEM((NS * 32,), jnp.int32),
          pltpu.VMEM((NS * 32,), jnp.int32),
          pltpu.VMEM((CHUNK,), jnp.int32),
          pltpu.VMEM((CHUNK,), jnp.int32),
          pltpu.VMEM((CHUNK, D), jnp.float32),
          pltpu.VMEM_SHARED((N_NODES, D), jnp.float32),
          pltpu.SemaphoreType.DMA,
      ],
  )
  def k(h_hbm, srcb_hbm, dstb_hbm, bases_hbm, trips_hbm, zero_hbm, out_hbm,
        bt_m, tt_m, si_m, di_m, rows_m, agg_sh, sem):
    c = lax.axis_index("c")
    s = lax.axis_index("s")
    wid = c * NS + s
    iota = lax.iota(jnp.int32, 16)

    pltpu.sync_copy(bases_hbm, bt_m)
    pltpu.sync_copy(trips_hbm, tt_m)

    r0 = wid * RANGE
    pltpu.sync_copy(zero_hbm.at[pl.ds(r0, RANGE)],
                    agg_sh.at[pl.ds(r0, RANGE)])

    @pl.when(wid == NW - 1)
    def _():
      pltpu.sync_copy(zero_hbm.at[pl.ds(NW * RANGE, TAIL)],
                      agg_sh.at[pl.ds(NW * RANGE, TAIL)])

    plsc.subcore_barrier()

    toff = pl.multiple_of(wid * NS, 8)
    bvec = bt_m[pl.ds(toff, 16)]
    tvec = tt_m[pl.ds(toff, 16)]
    for t in range(NS):
      base_t = bvec[t]
      trips_t = tvec[t]

      def run(g, _):
        cb = pl.multiple_of(base_t + g * CHUNK, 8)
        pltpu.sync_copy(srcb_hbm.at[pl.ds(cb, CHUNK)], si_m)
        pltpu.sync_copy(dstb_hbm.at[pl.ds(cb, CHUNK)], di_m)
        pltpu.async_copy(h_hbm.at[si_m], rows_m, sem).wait()
        pltpu.sync_copy(rows_m, agg_sh.at[di_m], add=True)
        return ()

      lax.fori_loop(0, trips_t, run, (), unroll=False)

    plsc.subcore_barrier()
    pltpu.sync_copy(agg_sh.at[pl.ds(r0, RANGE)],
                    out_hbm.at[pl.ds(r0, RANGE)])

    @pl.when(wid == NW - 1)
    def _():
      pltpu.sync_copy(agg_sh.at[pl.ds(NW * RANGE, TAIL)],
                      out_hbm.at[pl.ds(NW * RANGE, TAIL)])

  return k(h_pad, srcb, dstb, bases, trips, zeros)


_BLK = 2000
_PREC = lax.Precision.DEFAULT


def _dot(a, b):
  return lax.dot_general(a, b, (((1,), (0,)), ((), ())), precision=_PREC,
                         preferred_element_type=jnp.float32)


def _in_proj(x, w, bias):
  def body(x_ref, w_ref, b_ref, o_ref):
    o_ref[...] = _dot(x_ref[...], w_ref[...]) + b_ref[...]

  return pl.pallas_call(
      body,
      grid=(N_NODES // _BLK,),
      in_specs=[
          pl.BlockSpec((_BLK, D), lambda i: (i, 0)),
          pl.BlockSpec((D, D), lambda i: (0, 0)),
          pl.BlockSpec((1, D), lambda i: (0, 0)),
      ],
      out_specs=pl.BlockSpec((_BLK, D), lambda i: (i, 0)),
      out_shape=jax.ShapeDtypeStruct((N_NODES, D), jnp.float32),
  )(x, w, bias)


def _gin_update(h, agg, w, bias):
  def body(h_ref, a_ref, w_ref, b_ref, o_ref):
    o_ref[...] = _dot(h_ref[...] + a_ref[...], w_ref[...]) + b_ref[...]

  return pl.pallas_call(
      body,
      grid=(N_NODES // _BLK,),
      in_specs=[
          pl.BlockSpec((_BLK, D), lambda i: (i, 0)),
          pl.BlockSpec((_BLK, D), lambda i: (i, 0)),
          pl.BlockSpec((D, D), lambda i: (0, 0)),
          pl.BlockSpec((1, D), lambda i: (0, 0)),
      ],
      out_specs=pl.BlockSpec((_BLK, D), lambda i: (i, 0)),
      out_shape=jax.ShapeDtypeStruct((N_NODES, D), jnp.float32),
  )(h, agg, w, bias)


def _out_proj(h0, h1, h2, h3, w, bias):
  def body(h0_ref, h1_ref, h2_ref, h3_ref, w_ref, b_ref, o_ref):
    cat = jnp.concatenate(
        [h0_ref[...], h1_ref[...], h2_ref[...], h3_ref[...]], axis=1)
    logits = _dot(cat, w_ref[...]) + b_ref[...]
    m = jnp.max(logits, axis=-1, keepdims=True)
    e = jnp.exp(logits - m)
    o_ref[...] = e / jnp.sum(e, axis=-1, keepdims=True)

  hspec = pl.BlockSpec((_BLK, D), lambda i: (i, 0))
  return pl.pallas_call(
      body,
      grid=(N_NODES // _BLK,),
      in_specs=[
          hspec, hspec, hspec, hspec,
          pl.BlockSpec((4 * D, D), lambda i: (0, 0)),
          pl.BlockSpec((1, D), lambda i: (0, 0)),
      ],
      out_specs=hspec,
      out_shape=jax.ShapeDtypeStruct((N_NODES, D), jnp.float32),
  )(h0, h1, h2, h3, w, bias)


def kernel(x, edge_index, W_in, b_in, W1, b1, W2, b2, W3, b3, W_out, b_out):
  src = edge_index[0].astype(jnp.int32)
  dst = edge_index[1].astype(jnp.int32)
  zeros = jnp.zeros((N_NODES, D), jnp.float32)
  zrows = jnp.zeros((8, D), jnp.float32)

  srcb, dstb, bases, trips = _bin_edges(src, dst)
  bases = bases.reshape(NS, 32).T.reshape(-1)
  trips = trips.reshape(NS, 32).T.reshape(-1)

  h = _in_proj(x, W_in, b_in.reshape(1, D))
  hs = [h]
  for w, bias in ((W1, b1), (W2, b2), (W3, b3)):
    h_pad = jnp.concatenate([h, zrows], axis=0)
    agg = _sc_scatter(h_pad, srcb, dstb, bases, trips, zeros)
    h = _gin_update(h, agg, w, bias.reshape(1, D))
    hs.append(h)

  return _out_proj(hs[0], hs[1], hs[2], hs[3], W_out, b_out.reshape(1, D))
